# baseline (speedup 1.0000x reference)
import functools

import jax
import jax.numpy as jnp
from jax import lax
from jax.experimental import pallas as pl
from jax.experimental.pallas import tpu as pltpu
from jax.experimental.pallas import tpu_sc as plsc

_NC = 2
_NS = 16
_LANES = 16

_CHUNK = 64


def _sc_counts(dst2d, acc_rows, n_chunks):
  mesh = plsc.VectorSubcoreMesh(core_axis_name="c", subcore_axis_name="s")
  nz = acc_rows // _NS

  @functools.partial(
      pl.kernel,
      mesh=mesh,
      out_type=jax.ShapeDtypeStruct((_NC, acc_rows), jnp.float32),
      compiler_params=pltpu.CompilerParams(needs_layout_passes=False),
      scratch_types=[
          pltpu.VMEM((n_chunks, _CHUNK), jnp.int32),
          pltpu.VMEM((acc_rows,), jnp.float32),
          pltpu.VMEM((_NS, nz), jnp.float32),
          pltpu.VMEM((nz,), jnp.float32),
          pltpu.VMEM_SHARED((_NS, acc_rows), jnp.float32),
      ],
  )
  def count_kernel(dst_hbm, out_hbm, dst_v, cnt_v, red_v, res_v, cnt_sh):
    c = lax.axis_index("c")
    s = lax.axis_index("s")
    wid = c * _NS + s

    pltpu.sync_copy(dst_hbm.at[pl.ds(wid * n_chunks, n_chunks)], dst_v)

    zf = jnp.zeros((_LANES,), jnp.float32)

    def zero_body(i, carry):
      cnt_v[pl.ds(i * _LANES, _LANES)] = zf
      return carry

    lax.fori_loop(0, acc_rows // _LANES, zero_body, 0)

    def hist_body(k, carry):
      for j in range(_CHUNK // _LANES):
        idx = dst_v[k, pl.ds(j * _LANES, _LANES)]
        c16, last = plsc.scan_count(idx)
        plsc.addupdate_scatter(cnt_v, [idx], c16.astype(jnp.float32),
                               mask=last)
      return carry

    lax.fori_loop(0, n_chunks, hist_body, 0)

    pltpu.sync_copy(cnt_v, cnt_sh.at[s])
    plsc.subcore_barrier()
    pltpu.sync_copy(cnt_sh.at[:, pl.ds(s * nz, nz)], red_v)

    def red_body(g, carry):
      acc = jnp.zeros((_LANES,), jnp.float32)
      for r in range(_NS):
        acc = acc + red_v[r, pl.ds(g * _LANES, _LANES)]
      res_v[pl.ds(g * _LANES, _LANES)] = acc
      return carry

    lax.fori_loop(0, nz // _LANES, red_body, 0)
    pltpu.sync_copy(res_v, out_hbm.at[c, pl.ds(s * nz, nz)])

  return count_kernel(dst2d)


def _sc_segment_mean(src2d, dst2d, x, cnt, acc_rows, d_in, n_chunks):
  mesh = plsc.VectorSubcoreMesh(core_axis_name="c", subcore_axis_name="s")
  nz = acc_rows // _NS

  @functools.partial(
      pl.kernel,
      mesh=mesh,
      out_type=jax.ShapeDtypeStruct((_NC, acc_rows, d_in), jnp.float32),
      compiler_params=pltpu.CompilerParams(needs_layout_passes=False),
      scratch_types=[
          pltpu.VMEM((n_chunks // 2, _CHUNK), jnp.int32),
          pltpu.VMEM((n_chunks // 2, _CHUNK), jnp.int32),
          pltpu.VMEM((_CHUNK, d_in), jnp.float32),
          pltpu.VMEM((_CHUNK, d_in), jnp.float32),
          pltpu.VMEM((nz,), jnp.float32),
          pltpu.VMEM((nz,), jnp.float32),
          pltpu.VMEM_SHARED((acc_rows, d_in), jnp.float32),
          pltpu.SemaphoreType.DMA,
          pltpu.SemaphoreType.DMA,
      ],
  )
  def seg_mean(src_hbm, dst_hbm, x_hbm, cnt_hbm, out_hbm,
               src_v, dst_v, rows0_v, rows1_v, tmp_v, inv_v, acc_sh,
               sem0, sem1):
    c = lax.axis_index("c")
    s = lax.axis_index("s")
    wid = c * _NS + s

    pltpu.sync_copy(cnt_hbm.at[0, pl.ds(s * nz, nz)], tmp_v)
    pltpu.sync_copy(cnt_hbm.at[1, pl.ds(s * nz, nz)], inv_v)

    def inv_body(g, carry):
      tot = tmp_v[pl.ds(g * _LANES, _LANES)] + inv_v[pl.ds(g * _LANES, _LANES)]
      inv_v[pl.ds(g * _LANES, _LANES)] = 1.0 / jnp.maximum(tot, 1.0)
      return carry

    lax.fori_loop(0, nz // _LANES, inv_body, 0)

    zf = jnp.zeros((_LANES,), jnp.float32)
    for i in range(_CHUNK):
      for j in range(d_in // _LANES):
        rows0_v[i, pl.ds(j * _LANES, _LANES)] = zf

    def zero_acc_body(r, carry):
      pltpu.sync_copy(rows0_v, acc_sh.at[pl.ds(s * nz + r * _CHUNK, _CHUNK)])
      return carry

    lax.fori_loop(0, nz // _CHUNK, zero_acc_body, 0)
    plsc.subcore_barrier()

    half = n_chunks // 2
    for p in range(2):
      pltpu.sync_copy(src_hbm.at[pl.ds(wid * n_chunks + p * half, half)],
                      src_v)
      pltpu.sync_copy(dst_hbm.at[pl.ds(wid * n_chunks + p * half, half)],
                      dst_v)
      nhalf = half // 2
      pltpu.async_copy(x_hbm.at[src_v.at[0]], rows0_v, sem0)

      def body(k2, carry):
        k = 2 * k2
        pltpu.async_copy(x_hbm.at[src_v.at[k + 1]], rows1_v, sem1)
        pltpu.make_async_copy(x_hbm.at[src_v.at[k]], rows0_v, sem0).wait()
        pltpu.sync_copy(rows0_v, acc_sh.at[dst_v.at[k]], add=True)

        @pl.when(k2 + 1 < nhalf)
        def _():
          pltpu.async_copy(x_hbm.at[src_v.at[k + 2]], rows0_v, sem0)

        pltpu.make_async_copy(x_hbm.at[src_v.at[k + 1]], rows1_v, sem1).wait()
        pltpu.sync_copy(rows1_v, acc_sh.at[dst_v.at[k + 1]], add=True)
        return carry

      lax.fori_loop(0, nhalf, body, 0)
    plsc.subcore_barrier()

    def div_body(ch, carry):
      row0 = s * nz + ch * _CHUNK
      pltpu.sync_copy(acc_sh.at[pl.ds(row0, _CHUNK)], rows0_v)
      for g in range(_CHUNK // _LANES):
        inv16 = inv_v[pl.ds(ch * _CHUNK + g * _LANES, _LANES)]
        for l in range(_LANES):
          spl = jnp.take_along_axis(
              inv16, jnp.full((_LANES,), l, jnp.int32), axis=0)
          r = g * _LANES + l
          for cg in range(d_in // _LANES):
            rows0_v[r, pl.ds(cg * _LANES, _LANES)] = (
                rows0_v[r, pl.ds(cg * _LANES, _LANES)] * spl)
      pltpu.sync_copy(rows0_v, out_hbm.at[c, pl.ds(row0, _CHUNK)])
      return carry

    lax.fori_loop(0, nz // _CHUNK, div_body, 0)

  return seg_mean(src2d, dst2d, x, cnt)


def _tc_root(x, root, bias2d, n_nodes, d_in, d_hid):
  blk = 1000

  def body(x_ref, r_ref, b_ref, o_ref):
    o_ref[...] = jnp.dot(x_ref[...], r_ref[...],
                         preferred_element_type=jnp.float32) + b_ref[...]

  return pl.pallas_call(
      body,
      grid=(n_nodes // blk,),
      in_specs=[
          pl.BlockSpec((blk, d_in), lambda i: (i, 0)),
          pl.BlockSpec((d_in, d_hid), lambda i: (0, 0)),
          pl.BlockSpec((1, d_hid), lambda i: (0, 0)),
      ],
      out_specs=pl.BlockSpec((blk, d_hid), lambda i: (i, 0)),
      out_shape=jax.ShapeDtypeStruct((n_nodes, d_hid), jnp.float32),
  )(x, root, bias2d)


def _tc_finish(xr, partials, w0, n_nodes, d_in, d_hid):
  blk = 1000

  def body(xr_ref, p_ref, w_ref, o_ref):
    mean = p_ref[0] + p_ref[1]
    y = jnp.dot(mean, w_ref[...], preferred_element_type=jnp.float32)
    o_ref[...] = jnp.maximum(y + xr_ref[...], 0.0)

  return pl.pallas_call(
      body,
      grid=(n_nodes // blk,),
      in_specs=[
          pl.BlockSpec((blk, d_hid), lambda i: (i, 0)),
          pl.BlockSpec((_NC, blk, d_in), lambda i: (0, i, 0)),
          pl.BlockSpec((d_in, d_hid), lambda i: (0, 0)),
      ],
      out_specs=pl.BlockSpec((blk, d_hid), lambda i: (i, 0)),
      out_shape=jax.ShapeDtypeStruct((n_nodes, d_hid), jnp.float32),
  )(xr, partials, w0)


def kernel(x, edge_index, edge_attr, weight, root, bias):
  n_nodes, d_in = x.shape
  d_hid = root.shape[1]
  n_edges = edge_index.shape[1]
  del edge_attr

  nw = _NC * _NS
  n_chunks = -(-n_edges // (nw * _CHUNK * 8)) * 8
  e_pad = nw * n_chunks * _CHUNK
  pad = e_pad - n_edges

  nz = -(-(n_nodes + 1) // (_NS * _CHUNK)) * _CHUNK
  acc_rows = nz * _NS

  src = edge_index[0].astype(jnp.int32)
  dst = edge_index[1].astype(jnp.int32)
  if pad:
    ar = jnp.arange(pad, dtype=jnp.int32)
    src = jnp.concatenate([src, (ar * 97) % n_nodes])
    dst = jnp.concatenate([dst, n_nodes + ar % (acc_rows - n_nodes)])
  src2d = src.reshape(nw * n_chunks, _CHUNK)
  dst2d = dst.reshape(nw * n_chunks, _CHUNK)

  xr = _tc_root(x, root, bias.reshape(1, d_hid), n_nodes, d_in, d_hid)
  cnt = _sc_counts(dst2d, acc_rows, n_chunks)
  partials = _sc_segment_mean(src2d, dst2d, x, cnt, acc_rows, d_in, n_chunks)
  return _tc_finish(xr, partials, weight[0], n_nodes, d_in, d_hid)


# speedup vs baseline: 1.0830x; 1.0830x over previous
import functools

import jax
import numpy as np
import jax.numpy as jnp
from jax import lax
from jax.experimental import pallas as pl
from jax.experimental.pallas import tpu as pltpu
from jax.experimental.pallas import tpu_sc as plsc

_NC = 2
_NS = 16
_LANES = 16

_CHUNK = 64


def _sc_counts(e3d, acc_rows, n_chunks):
  mesh = plsc.VectorSubcoreMesh(core_axis_name="c", subcore_axis_name="s")
  nz = acc_rows // _NS

  @functools.partial(
      pl.kernel,
      mesh=mesh,
      out_type=jax.ShapeDtypeStruct((_NC, acc_rows), jnp.float32),
      compiler_params=pltpu.CompilerParams(needs_layout_passes=False),
      scratch_types=[
          pltpu.VMEM((n_chunks, _CHUNK), jnp.int32),
          pltpu.VMEM((acc_rows,), jnp.float32),
          pltpu.VMEM((_NS, nz), jnp.float32),
          pltpu.VMEM((nz,), jnp.float32),
          pltpu.VMEM_SHARED((_NS, acc_rows), jnp.float32),
      ],
  )
  def count_kernel(e_hbm, out_hbm, dst_v, cnt_v, red_v, res_v, cnt_sh):
    c = lax.axis_index("c")
    s = lax.axis_index("s")
    wid = c * _NS + s

    pltpu.sync_copy(e_hbm.at[1, pl.ds(wid * n_chunks, n_chunks)], dst_v)

    zf = jnp.zeros((_LANES,), jnp.float32)

    def zero_body(i, carry):
      cnt_v[pl.ds(i * _LANES, _LANES)] = zf
      return carry

    lax.fori_loop(0, acc_rows // _LANES, zero_body, 0)

    def hist_body(k, carry):
      for j in range(_CHUNK // _LANES):
        idx = dst_v[k, pl.ds(j * _LANES, _LANES)]
        c16, last = plsc.scan_count(idx)
        plsc.addupdate_scatter(cnt_v, [idx], c16.astype(jnp.float32),
                               mask=last)
      return carry

    lax.fori_loop(0, n_chunks, hist_body, 0)

    pltpu.sync_copy(cnt_v, cnt_sh.at[s])
    plsc.subcore_barrier()
    pltpu.sync_copy(cnt_sh.at[:, pl.ds(s * nz, nz)], red_v)

    def red_body(g, carry):
      acc = jnp.zeros((_LANES,), jnp.float32)
      for r in range(_NS):
        acc = acc + red_v[r, pl.ds(g * _LANES, _LANES)]
      res_v[pl.ds(g * _LANES, _LANES)] = acc
      return carry

    lax.fori_loop(0, nz // _LANES, red_body, 0)
    pltpu.sync_copy(res_v, out_hbm.at[c, pl.ds(s * nz, nz)])

  return count_kernel(e3d)


def _sc_segment_mean(e3d, x, cnt, acc_rows, d_in, n_chunks):
  mesh = plsc.VectorSubcoreMesh(core_axis_name="c", subcore_axis_name="s")
  nz = acc_rows // _NS

  @functools.partial(
      pl.kernel,
      mesh=mesh,
      out_type=jax.ShapeDtypeStruct((_NC, acc_rows, d_in), jnp.float32),
      compiler_params=pltpu.CompilerParams(needs_layout_passes=False),
      scratch_types=[
          pltpu.VMEM((n_chunks // 2, _CHUNK), jnp.int32),
          pltpu.VMEM((n_chunks // 2, _CHUNK), jnp.int32),
          pltpu.VMEM((_CHUNK, d_in), jnp.float32),
          pltpu.VMEM((_CHUNK, d_in), jnp.float32),
          pltpu.VMEM((nz,), jnp.float32),
          pltpu.VMEM((nz,), jnp.float32),
          pltpu.VMEM_SHARED((acc_rows, d_in), jnp.float32),
          pltpu.SemaphoreType.DMA,
          pltpu.SemaphoreType.DMA,
      ],
  )
  def seg_mean(e_hbm, x_hbm, cnt_hbm, out_hbm,
               src_v, dst_v, rows0_v, rows1_v, tmp_v, inv_v, acc_sh,
               sem0, sem1):
    c = lax.axis_index("c")
    s = lax.axis_index("s")
    wid = c * _NS + s

    zf = jnp.zeros((_LANES,), jnp.float32)
    for i in range(_CHUNK):
      for j in range(d_in // _LANES):
        rows0_v[i, pl.ds(j * _LANES, _LANES)] = zf
    nzc = nz // _CHUNK
    for r in range(nzc):
      pltpu.async_copy(rows0_v, acc_sh.at[pl.ds(s * nz + r * _CHUNK, _CHUNK)],
                       sem0)

    pltpu.sync_copy(cnt_hbm.at[0, pl.ds(s * nz, nz)], tmp_v)
    pltpu.sync_copy(cnt_hbm.at[1, pl.ds(s * nz, nz)], inv_v)

    def inv_body(g, carry):
      tot = tmp_v[pl.ds(g * _LANES, _LANES)] + inv_v[pl.ds(g * _LANES, _LANES)]
      inv_v[pl.ds(g * _LANES, _LANES)] = 1.0 / jnp.maximum(tot, 1.0)
      return carry

    lax.fori_loop(0, nz // _LANES, inv_body, 0)

    for r in range(nzc):
      pltpu.make_async_copy(
          rows0_v, acc_sh.at[pl.ds(s * nz + r * _CHUNK, _CHUNK)], sem0).wait()
    plsc.subcore_barrier()

    half = n_chunks // 2
    for p in range(2):
      pltpu.sync_copy(e_hbm.at[0, pl.ds(wid * n_chunks + p * half, half)],
                      src_v)
      pltpu.sync_copy(e_hbm.at[1, pl.ds(wid * n_chunks + p * half, half)],
                      dst_v)
      nhalf = half // 2
      pltpu.async_copy(x_hbm.at[src_v.at[0]], rows0_v, sem0)

      def body(k2, carry):
        k = 2 * k2
        pltpu.async_copy(x_hbm.at[src_v.at[k + 1]], rows1_v, sem1)
        pltpu.make_async_copy(x_hbm.at[src_v.at[k]], rows0_v, sem0).wait()
        pltpu.sync_copy(rows0_v, acc_sh.at[dst_v.at[k]], add=True)

        @pl.when(k2 + 1 < nhalf)
        def _():
          pltpu.async_copy(x_hbm.at[src_v.at[k + 2]], rows0_v, sem0)

        pltpu.make_async_copy(x_hbm.at[src_v.at[k + 1]], rows1_v, sem1).wait()
        pltpu.sync_copy(rows1_v, acc_sh.at[dst_v.at[k + 1]], add=True)
        return carry

      lax.fori_loop(0, nhalf, body, 0)
    plsc.subcore_barrier()

    nzc = nz // _CHUNK

    def _divide(buf, ch):
      for g in range(_CHUNK // _LANES):
        inv16 = inv_v[pl.ds(ch * _CHUNK + g * _LANES, _LANES)]
        for l in range(_LANES):
          spl = jnp.take_along_axis(
              inv16, jnp.full((_LANES,), l, jnp.int32), axis=0)
          r = g * _LANES + l
          for cg in range(d_in // _LANES):
            buf[r, pl.ds(cg * _LANES, _LANES)] = (
                buf[r, pl.ds(cg * _LANES, _LANES)] * spl)

    def _row0(ch):
      return s * nz + ch * _CHUNK

    pltpu.sync_copy(acc_sh.at[pl.ds(_row0(0), _CHUNK)], rows0_v)

    def div_body(chp, carry):
      ch = 2 * chp
      pltpu.async_copy(acc_sh.at[pl.ds(_row0(ch + 1), _CHUNK)], rows1_v, sem1)
      _divide(rows0_v, ch)
      pltpu.async_copy(rows0_v, out_hbm.at[c, pl.ds(_row0(ch), _CHUNK)], sem0)
      pltpu.make_async_copy(
          acc_sh.at[pl.ds(_row0(ch + 1), _CHUNK)], rows1_v, sem1).wait()
      _divide(rows1_v, ch + 1)
      pltpu.async_copy(rows1_v, out_hbm.at[c, pl.ds(_row0(ch + 1), _CHUNK)],
                       sem1)
      pltpu.make_async_copy(
          rows0_v, out_hbm.at[c, pl.ds(_row0(ch), _CHUNK)], sem0).wait()

      @pl.when(chp + 1 < nzc // 2)
      def _():
        pltpu.sync_copy(acc_sh.at[pl.ds(_row0(ch + 2), _CHUNK)], rows0_v)

      pltpu.make_async_copy(
          rows1_v, out_hbm.at[c, pl.ds(_row0(ch + 1), _CHUNK)], sem1).wait()
      return carry

    lax.fori_loop(0, nzc // 2, div_body, 0)

  return seg_mean(e3d, x, cnt)


def _tc_root(x, root, bias2d, n_nodes, d_in, d_hid):
  blk = 1000

  def body(x_ref, r_ref, b_ref, o_ref):
    o_ref[...] = jnp.dot(x_ref[...], r_ref[...],
                         preferred_element_type=jnp.float32) + b_ref[...]

  return pl.pallas_call(
      body,
      grid=(n_nodes // blk,),
      in_specs=[
          pl.BlockSpec((blk, d_in), lambda i: (i, 0)),
          pl.BlockSpec((d_in, d_hid), lambda i: (0, 0)),
          pl.BlockSpec((1, d_hid), lambda i: (0, 0)),
      ],
      out_specs=pl.BlockSpec((blk, d_hid), lambda i: (i, 0)),
      out_shape=jax.ShapeDtypeStruct((n_nodes, d_hid), jnp.float32),
  )(x, root, bias2d)


def _tc_finish(xr, partials, w0, n_nodes, d_in, d_hid):
  blk = 1000

  def body(xr_ref, p_ref, w_ref, o_ref):
    mean = p_ref[0] + p_ref[1]
    y = jnp.dot(mean, w_ref[...], preferred_element_type=jnp.float32)
    o_ref[...] = jnp.maximum(y + xr_ref[...], 0.0)

  return pl.pallas_call(
      body,
      grid=(n_nodes // blk,),
      in_specs=[
          pl.BlockSpec((blk, d_hid), lambda i: (i, 0)),
          pl.BlockSpec((_NC, blk, d_in), lambda i: (0, i, 0)),
          pl.BlockSpec((d_in, d_hid), lambda i: (0, 0)),
      ],
      out_specs=pl.BlockSpec((blk, d_hid), lambda i: (i, 0)),
      out_shape=jax.ShapeDtypeStruct((n_nodes, d_hid), jnp.float32),
  )(xr, partials, w0)


def kernel(x, edge_index, edge_attr, weight, root, bias):
  n_nodes, d_in = x.shape
  d_hid = root.shape[1]
  n_edges = edge_index.shape[1]
  del edge_attr

  nw = _NC * _NS
  n_chunks = -(-n_edges // (nw * _CHUNK * 8)) * 8
  e_pad = nw * n_chunks * _CHUNK
  pad = e_pad - n_edges

  nz = -(-(n_nodes + 1) // (_NS * _CHUNK)) * _CHUNK
  acc_rows = nz * _NS

  e3d = edge_index.astype(jnp.int32).reshape(2, n_edges // _CHUNK, _CHUNK)
  if pad:
    ar = np.arange(pad, dtype=np.int32)
    pad_blk = jnp.asarray(
        np.stack([(ar * 97) % n_nodes,
                  n_nodes + ar % (acc_rows - n_nodes)]).reshape(
                      2, pad // _CHUNK, _CHUNK))
    e3d = jnp.concatenate([e3d, pad_blk], axis=1)

  xr = _tc_root(x, root, bias.reshape(1, d_hid), n_nodes, d_in, d_hid)
  cnt = _sc_counts(e3d, acc_rows, n_chunks)
  partials = _sc_segment_mean(e3d, x, cnt, acc_rows, d_in, n_chunks)
  return _tc_finish(xr, partials, weight[0], n_nodes, d_in, d_hid)
